# Initial kernel scaffold; baseline (speedup 1.0000x reference)
import jax
import jax.numpy as jnp
from jax.experimental import pallas as pl


def kernel(x, edge_index, edge_attr, batch, W_lin0, b_lin0, W_e1, b_e1, W_e2, b_e2, W_root, b_conv, W_ih_gru, W_hh_gru, b_ih_gru, b_hh_gru, W_ih_s2s, W_hh_s2s, b_ih_s2s, b_hh_s2s, W_ih_mem, W_hh_mem, b_ih_mem, b_hh_mem, W_lin1, b_lin1, W_lin3, b_lin3):
    raise NotImplementedError("write your pallas kernel here")


# speedup vs baseline: 3.1066x; 3.1066x over previous
import jax
import jax.numpy as jnp
from jax.experimental import pallas as pl
from jax.experimental.pallas import tpu as pltpu

D = 64
N_NODES = 5000
N_EDGES = 10000
NPAD = 5120
EPAD = 10240
ET = 512
NT = EPAD // ET

_f32 = jnp.float32

_DNN = (((1,), (0,)), ((), ()))
_DNT = (((1,), (1,)), ((), ()))


def _mm(a, b):
    return jax.lax.dot_general(a, b, _DNN, preferred_element_type=_f32)


def _mmt(a, b):
    return jax.lax.dot_general(a, b, _DNT, preferred_element_type=_f32)


def _sigmoid(x):
    return jax.nn.sigmoid(x)


def _net_kernel(xT_ref, src3_ref, dst3_ref, eaT_ref,
                Wl0T_ref, bl0_ref, We1T_ref, be1_ref, W2pT_ref, B2T_ref,
                WrT_ref, bcv_ref, WigT_ref, WhgT_ref, big_ref, bhg_ref,
                WisT_ref, WhsT_ref, bis_ref, bhs_ref,
                WimT_ref, WhmT_ref, bim_ref, bhm_ref,
                Wl1T_ref, bl1_ref, Wl3T_ref, bl3_ref,
                v_ref, hx_ref, cx_ref,
                hT_ref, aggT_ref, hidT3_ref, UT_ref, rdeg_ref):
    hT_ref[:] = jax.nn.relu(_mm(Wl0T_ref[:], xT_ref[:]) + bl0_ref[:])

    eaT = eaT_ref[:]
    for i in range(NT):
        hidT3_ref[i] = jax.nn.relu(
            _mm(We1T_ref[:], eaT[:, i * ET:(i + 1) * ET]) + be1_ref[:])

    rdeg_ref[:] = jnp.zeros((1, NPAD), _f32)
    ones_row = jnp.ones((1, ET), _f32)

    def deg_body(i, c):
        dst_row = dst3_ref[i]
        niota = jax.lax.broadcasted_iota(jnp.int32, (NPAD, ET), 0)
        oh = jnp.where(dst_row == niota, 1.0, 0.0)
        rdeg_ref[:] = rdeg_ref[:] + _mmt(ones_row, oh)
        return c

    jax.lax.fori_loop(0, NT, deg_body, 0)
    rdeg_ref[:] = 1.0 / jnp.maximum(rdeg_ref[:], 1.0)

    def iter_body(t, c):
        aggT_ref[:] = jnp.zeros((D, NPAD), _f32)

        def edge_body(i, cc):
            src_row = src3_ref[i]
            niota = jax.lax.broadcasted_iota(jnp.int32, (NPAD, ET), 0)
            ohG = jnp.where(src_row == niota, 1.0, 0.0)
            out_srcT = _mm(hT_ref[:], ohG)
            UT_ref[:] = _mm(W2pT_ref[:], out_srcT)
            hidT = hidT3_ref[i]
            msgT = _mm(B2T_ref[:], out_srcT)
            for k in range(D):
                msgT = msgT + hidT[k:k + 1, :] * UT_ref[k * D:(k + 1) * D, :]
            dst_row = dst3_ref[i]
            ohS = jnp.where(dst_row == niota, 1.0, 0.0)
            aggT_ref[:] = aggT_ref[:] + _mmt(msgT, ohS)
            return cc

        jax.lax.fori_loop(0, NT, edge_body, 0)

        mT = jax.nn.relu(aggT_ref[:] * rdeg_ref[:]
                         + _mm(WrT_ref[:], hT_ref[:]) + bcv_ref[:])
        giT = _mm(WigT_ref[:], mT) + big_ref[:]
        ghT = _mm(WhgT_ref[:], hT_ref[:]) + bhg_ref[:]
        r = _sigmoid(giT[0:D, :] + ghT[0:D, :])
        z = _sigmoid(giT[D:2 * D, :] + ghT[D:2 * D, :])
        cand = jnp.tanh(giT[2 * D:3 * D, :] + r * ghT[2 * D:3 * D, :])
        hT_ref[:] = (1.0 - z) * cand + z * hT_ref[:]
        return c

    jax.lax.fori_loop(0, 6, iter_body, 0)

    def lstm(xT, hsT, csT, WiT, WhT, bi, bh):
        g = _mm(WiT, xT) + bi + _mm(WhT, hsT) + bh
        ii = _sigmoid(g[0:D, :])
        ff = _sigmoid(g[D:2 * D, :])
        gg = jnp.tanh(g[2 * D:3 * D, :])
        oo = _sigmoid(g[3 * D:4 * D, :])
        cs = ff * csT + ii * gg
        hs = oo * jnp.tanh(cs)
        return hs, cs

    outT = hT_ref[:]
    lanemask = jax.lax.broadcasted_iota(jnp.int32, (1, NPAD), 1) < N_NODES
    q_starT = jnp.zeros((2 * D, 1), _f32)
    hsT = jnp.zeros((D, 1), _f32)
    csT = jnp.zeros((D, 1), _f32)
    for _ in range(6):
        hsT, csT = lstm(q_starT, hsT, csT, WisT_ref[:], WhsT_ref[:],
                        bis_ref[:], bhs_ref[:])
        qT = hsT
        e_row = jnp.sum(outT * qT, axis=0, keepdims=True)
        e_row = jnp.where(lanemask, e_row, -1e30)
        mx = jnp.max(e_row, axis=1, keepdims=True)
        a_row = jnp.exp(e_row - mx)
        a_row = jnp.where(lanemask, a_row, 0.0)
        s = jnp.sum(a_row, axis=1, keepdims=True)
        a_row = a_row / s
        rvecT = jnp.sum(outT * a_row, axis=1, keepdims=True)
        q_starT = jnp.concatenate([qT, rvecT], axis=0)

    hxT, cxT = lstm(q_starT, jnp.zeros((D, 1), _f32), jnp.zeros((D, 1), _f32),
                    WimT_ref[:], WhmT_ref[:], bim_ref[:], bhm_ref[:])
    o1T = jax.nn.relu(_mm(Wl1T_ref[:], hxT) + bl1_ref[:])
    vT = _mm(Wl3T_ref[:], o1T) + bl3_ref[:]

    v_ref[:] = vT
    hx_ref[:] = hxT
    cx_ref[:] = cxT


def kernel(x, edge_index, edge_attr, batch, W_lin0, b_lin0, W_e1, b_e1,
           W_e2, b_e2, W_root, b_conv, W_ih_gru, W_hh_gru, b_ih_gru, b_hh_gru,
           W_ih_s2s, W_hh_s2s, b_ih_s2s, b_hh_s2s, W_ih_mem, W_hh_mem,
           b_ih_mem, b_hh_mem, W_lin1, b_lin1, W_lin3, b_lin3):
    xT = jnp.zeros((8, NPAD), _f32).at[:3, :N_NODES].set(x.T)
    src = edge_index[0].astype(jnp.int32)
    dst = edge_index[1].astype(jnp.int32)
    src_p = jnp.full((EPAD,), 0, jnp.int32).at[:N_EDGES].set(src)
    dst_p = jnp.full((EPAD,), NPAD - 1, jnp.int32).at[:N_EDGES].set(dst)
    src3 = src_p.reshape(NT, 1, ET)
    dst3 = dst_p.reshape(NT, 1, ET)
    eaT = jnp.zeros((8, EPAD), _f32).at[:7, :N_EDGES].set(edge_attr.T)

    col = lambda b: b.reshape(-1, 1).astype(_f32)
    Wl0T = jnp.zeros((D, 8), _f32).at[:, :3].set(W_lin0.T)
    We1T = jnp.zeros((D, 8), _f32).at[:, :7].set(W_e1.T)
    W2pT = W_e2.reshape(D, D, D).transpose(0, 2, 1).reshape(D * D, D)
    B2T = b_e2.reshape(D, D).T

    outs = pl.pallas_call(
        _net_kernel,
        out_shape=[
            jax.ShapeDtypeStruct((1, 1), _f32),
            jax.ShapeDtypeStruct((D, 1), _f32),
            jax.ShapeDtypeStruct((D, 1), _f32),
        ],
        scratch_shapes=[
            pltpu.VMEM((D, NPAD), _f32),
            pltpu.VMEM((D, NPAD), _f32),
            pltpu.VMEM((NT, D, ET), _f32),
            pltpu.VMEM((D * D, ET), _f32),
            pltpu.VMEM((1, NPAD), _f32),
        ],
    )(xT, src3, dst3, eaT,
      Wl0T, col(b_lin0), We1T, col(b_e1),
      W2pT, B2T,
      W_root.T.astype(_f32), col(b_conv),
      W_ih_gru.T.astype(_f32), W_hh_gru.T.astype(_f32),
      col(b_ih_gru), col(b_hh_gru),
      W_ih_s2s.T.astype(_f32), W_hh_s2s.T.astype(_f32),
      col(b_ih_s2s), col(b_hh_s2s),
      W_ih_mem.T.astype(_f32), W_hh_mem.T.astype(_f32),
      col(b_ih_mem), col(b_hh_mem),
      W_lin1.T.astype(_f32), col(b_lin1), W_lin3.T.astype(_f32), col(b_lin3))

    v, hxT, cxT = outs
    hx = hxT.reshape(1, 1, D)
    cx = cxT.reshape(1, 1, D)
    return (v, hx, cx)
